# Initial kernel scaffold; baseline (speedup 1.0000x reference)
import jax
import jax.numpy as jnp
from jax.experimental import pallas as pl


def kernel(fatoms, fbonds, agraph, bgraph, tree_message, atom_scope, W_i, W_h, W_o_w, W_o_b):
    raise NotImplementedError("write your pallas kernel here")


# speedup vs baseline: 3.3058x; 3.3058x over previous
import functools

import jax
import jax.numpy as jnp
from jax import lax
from jax.experimental import pallas as pl
from jax.experimental.pallas import tpu as pltpu
from jax.experimental.pallas import tpu_sc as plsc

H = 128
NB = 10
NW = 32
CH = 40
IG = 80
NG = (CH * NB) // IG
DEPTH = 3
N_MOLS = 500


def _gather_sum(table, idx2d, n_out):
    per_w = n_out // NW
    n_chunks = per_w // CH
    mesh = plsc.VectorSubcoreMesh(core_axis_name="c", subcore_axis_name="s")

    @functools.partial(
        pl.kernel,
        out_type=jax.ShapeDtypeStruct((n_out, H), jnp.float32),
        mesh=mesh,
        scratch_types=[
            pltpu.VMEM((NG, IG), jnp.int32),
            pltpu.VMEM((CH * NB, H), jnp.float32),
            pltpu.VMEM((CH, H), jnp.float32),
            pltpu.SemaphoreType.DMA,
        ],
    )
    def k(table_hbm, idx_hbm, out_hbm, idx_v, rows_v, out_v, sem):
        wid = lax.axis_index("s") * 2 + lax.axis_index("c")

        def chunk(ci, carry):
            base = wid * per_w + ci * CH
            pltpu.sync_copy(idx_hbm.at[base // CH], idx_v)
            copies = [
                pltpu.async_copy(
                    table_hbm.at[idx_v.at[g]],
                    rows_v.at[pl.ds(g * IG, IG)],
                    sem,
                )
                for g in range(NG)
            ]
            for cp in copies:
                cp.wait()

            def bond(i, carry2):
                r0 = i * NB
                for h in range(H // 16):
                    sl = pl.ds(h * 16, 16)
                    acc = rows_v[r0, sl]
                    for kk in range(1, NB):
                        acc = acc + rows_v[r0 + kk, sl]
                    out_v[i, sl] = acc
                return carry2

            lax.fori_loop(0, CH, bond, 0)
            pltpu.sync_copy(out_v, out_hbm.at[pl.ds(base, CH)])
            return carry

        lax.fori_loop(0, n_chunks, chunk, 0)

    return k(table, idx2d)


def _bond_input(fbonds, W_i):
    n = fbonds.shape[0]
    blk = 3200
    fd = fbonds.shape[1]

    def body(x_ref, w_ref, bin_ref, gm_ref):
        y = lax.dot_general(
            x_ref[...], w_ref[...], (((1,), (1,)), ((), ())),
            preferred_element_type=jnp.float32)
        bin_ref[...] = y
        gm_ref[...] = jnp.maximum(y, 0.0)

    return pl.pallas_call(
        body,
        grid=(n // blk,),
        in_specs=[
            pl.BlockSpec((blk, fd), lambda i: (i, 0)),
            pl.BlockSpec((H, fd), lambda i: (0, 0)),
        ],
        out_specs=[
            pl.BlockSpec((blk, H), lambda i: (i, 0)),
            pl.BlockSpec((blk, H), lambda i: (i, 0)),
        ],
        out_shape=[
            jax.ShapeDtypeStruct((n, H), jnp.float32),
            jax.ShapeDtypeStruct((n, H), jnp.float32),
        ],
    )(fbonds, W_i)


def _depth_update(nei, binput, W_h):
    n = nei.shape[0]
    blk = 3200

    def body(nei_ref, bin_ref, w_ref, out_ref):
        y = lax.dot_general(
            nei_ref[...], w_ref[...], (((1,), (1,)), ((), ())),
            preferred_element_type=jnp.float32)
        out_ref[...] = jnp.maximum(bin_ref[...] + y, 0.0)

    return pl.pallas_call(
        body,
        grid=(n // blk,),
        in_specs=[
            pl.BlockSpec((blk, H), lambda i: (i, 0)),
            pl.BlockSpec((blk, H), lambda i: (i, 0)),
            pl.BlockSpec((H, H), lambda i: (0, 0)),
        ],
        out_specs=pl.BlockSpec((blk, H), lambda i: (i, 0)),
        out_shape=jax.ShapeDtypeStruct((n, H), jnp.float32),
    )(nei, binput, W_h)


def _output_layer(fatoms, anei, scope_col, W_o1, W_o2, W_o_b):
    n = fatoms.shape[0]
    blk = 1000
    nblk = n // blk
    afd = fatoms.shape[1]

    def body(fa_ref, an_ref, sc_ref, w1_ref, w2_ref, b_ref, out_ref, cnt):
        i = pl.program_id(0)

        @pl.when(i == 0)
        def _():
            out_ref[...] = jnp.zeros_like(out_ref)
            cnt[...] = jnp.zeros_like(cnt)

        y = lax.dot_general(
            fa_ref[...], w1_ref[...], (((1,), (1,)), ((), ())),
            preferred_element_type=jnp.float32)
        y = y + lax.dot_general(
            an_ref[...], w2_ref[...], (((1,), (1,)), ((), ())),
            preferred_element_type=jnp.float32)
        hidden = jnp.maximum(y + b_ref[...], 0.0)
        ohT = (sc_ref[...] == lax.broadcasted_iota(
            jnp.int32, (blk, N_MOLS), 1)).astype(jnp.float32)
        out_ref[...] += lax.dot_general(
            ohT, hidden, (((0,), (0,)), ((), ())),
            preferred_element_type=jnp.float32)
        cnt[...] += lax.dot_general(
            ohT, jnp.ones_like(hidden), (((0,), (0,)), ((), ())),
            preferred_element_type=jnp.float32)

        @pl.when(i == nblk - 1)
        def _():
            out_ref[...] = out_ref[...] / jnp.maximum(cnt[...], 1.0)

    return pl.pallas_call(
        body,
        grid=(nblk,),
        in_specs=[
            pl.BlockSpec((blk, afd), lambda i: (i, 0)),
            pl.BlockSpec((blk, H), lambda i: (i, 0)),
            pl.BlockSpec((blk, 1), lambda i: (i, 0)),
            pl.BlockSpec((H, afd), lambda i: (0, 0)),
            pl.BlockSpec((H, H), lambda i: (0, 0)),
            pl.BlockSpec((1, H), lambda i: (0, 0)),
        ],
        out_specs=pl.BlockSpec((N_MOLS, H), lambda i: (0, 0)),
        out_shape=jax.ShapeDtypeStruct((N_MOLS, H), jnp.float32),
        scratch_shapes=[pltpu.VMEM((N_MOLS, H), jnp.float32)],
    )(fatoms, anei, scope_col, W_o1, W_o2, W_o_b)


def kernel(fatoms, fbonds, agraph, bgraph, tree_message, atom_scope,
           W_i, W_h, W_o_w, W_o_b):
    n_atoms = fatoms.shape[0]
    afd = fatoms.shape[1]

    bidx = bgraph.astype(jnp.int32).reshape(-1, NG, IG)
    n_atoms_pad = 10240
    aidx = jnp.pad(agraph.astype(jnp.int32),
                   ((0, n_atoms_pad - n_atoms), (0, 0))).reshape(-1, NG, IG)

    binput, gm = _bond_input(fbonds, W_i)
    for _ in range(DEPTH - 1):
        table = jnp.concatenate([tree_message, gm], axis=0)
        nei = _gather_sum(table, bidx, bgraph.shape[0])
        gm = _depth_update(nei, binput, W_h)

    table = jnp.concatenate([tree_message, gm], axis=0)
    anei = _gather_sum(table, aidx, n_atoms_pad)[:n_atoms]

    scope_col = atom_scope.astype(jnp.int32).reshape(-1, 1)
    W_o1 = W_o_w[:, :afd]
    W_o2 = W_o_w[:, afd:]
    b_row = W_o_b.reshape(1, H)
    return _output_layer(fatoms, anei, scope_col, W_o1, W_o2, b_row)


# speedup vs baseline: 4.4188x; 1.3367x over previous
import functools

import jax
import jax.numpy as jnp
from jax import lax
from jax.experimental import pallas as pl
from jax.experimental.pallas import tpu as pltpu
from jax.experimental.pallas import tpu_sc as plsc

H = 128
NB = 10
NW = 32
CH = 40
IG = 80
NG = (CH * NB) // IG
DEPTH = 3
N_MOLS = 500


def _gather_sum(table, idx2d, n_out):
    per_w = n_out // NW
    n_chunks = per_w // CH
    assert n_chunks % 2 == 0
    mesh = plsc.VectorSubcoreMesh(core_axis_name="c", subcore_axis_name="s")

    @functools.partial(
        pl.kernel,
        out_type=jax.ShapeDtypeStruct((n_out, H), jnp.float32),
        mesh=mesh,
        scratch_types=[
            pltpu.VMEM((2, NG, IG), jnp.int32),
            pltpu.VMEM((2, CH * NB, H), jnp.float32),
            pltpu.VMEM((CH, H), jnp.float32),
            pltpu.SemaphoreType.DMA,
            pltpu.SemaphoreType.DMA,
        ],
    )
    def k(table_hbm, idx_hbm, out_hbm, idx_v, rows_v, out_v, sem0, sem1):
        wid = lax.axis_index("s") * 2 + lax.axis_index("c")
        chunk0 = (wid * per_w) // CH
        sems = (sem0, sem1)

        def issue(ci, b):
            pltpu.sync_copy(idx_hbm.at[chunk0 + ci], idx_v.at[b])
            for g in range(NG):
                pltpu.async_copy(
                    table_hbm.at[idx_v.at[b, g]],
                    rows_v.at[b, pl.ds(g * IG, IG)],
                    sems[b],
                )

        def drain(b):
            pltpu.make_async_copy(
                table_hbm.at[pl.ds(0, CH * NB)], rows_v.at[b], sems[b]
            ).wait()

        def reduce_store(ci, b):
            def bond(i, carry2):
                r0 = i * NB
                for h in range(H // 16):
                    sl = pl.ds(h * 16, 16)
                    acc = rows_v[b, r0, sl]
                    for kk in range(1, NB):
                        acc = acc + rows_v[b, r0 + kk, sl]
                    out_v[i, sl] = acc
                return carry2

            lax.fori_loop(0, CH, bond, 0)
            pltpu.sync_copy(out_v, out_hbm.at[pl.ds(wid * per_w + ci * CH, CH)])

        issue(0, 0)

        def pair(j, carry):
            ci0 = 2 * j
            issue(ci0 + 1, 1)
            drain(0)
            reduce_store(ci0, 0)

            @pl.when(j < n_chunks // 2 - 1)
            def _():
                issue(ci0 + 2, 0)

            drain(1)
            reduce_store(ci0 + 1, 1)
            return carry

        lax.fori_loop(0, n_chunks // 2, pair, 0)

    return k(table, idx2d)


def _bond_input(fbonds, W_i):
    n = fbonds.shape[0]
    blk = 3200
    fd = fbonds.shape[1]

    def body(x_ref, w_ref, bin_ref, gm_ref):
        y = lax.dot_general(
            x_ref[...], w_ref[...], (((1,), (1,)), ((), ())),
            preferred_element_type=jnp.float32)
        bin_ref[...] = y
        gm_ref[...] = jnp.maximum(y, 0.0)

    return pl.pallas_call(
        body,
        grid=(n // blk,),
        in_specs=[
            pl.BlockSpec((blk, fd), lambda i: (i, 0)),
            pl.BlockSpec((H, fd), lambda i: (0, 0)),
        ],
        out_specs=[
            pl.BlockSpec((blk, H), lambda i: (i, 0)),
            pl.BlockSpec((blk, H), lambda i: (i, 0)),
        ],
        out_shape=[
            jax.ShapeDtypeStruct((n, H), jnp.float32),
            jax.ShapeDtypeStruct((n, H), jnp.float32),
        ],
    )(fbonds, W_i)


def _depth_update(nei, binput, W_h):
    n = nei.shape[0]
    blk = 3200

    def body(nei_ref, bin_ref, w_ref, out_ref):
        y = lax.dot_general(
            nei_ref[...], w_ref[...], (((1,), (1,)), ((), ())),
            preferred_element_type=jnp.float32)
        out_ref[...] = jnp.maximum(bin_ref[...] + y, 0.0)

    return pl.pallas_call(
        body,
        grid=(n // blk,),
        in_specs=[
            pl.BlockSpec((blk, H), lambda i: (i, 0)),
            pl.BlockSpec((blk, H), lambda i: (i, 0)),
            pl.BlockSpec((H, H), lambda i: (0, 0)),
        ],
        out_specs=pl.BlockSpec((blk, H), lambda i: (i, 0)),
        out_shape=jax.ShapeDtypeStruct((n, H), jnp.float32),
    )(nei, binput, W_h)


def _output_layer(fatoms, anei, scope_col, W_o1, W_o2, W_o_b):
    n = fatoms.shape[0]
    blk = 1000
    nblk = n // blk
    afd = fatoms.shape[1]

    def body(fa_ref, an_ref, sc_ref, w1_ref, w2_ref, b_ref, out_ref, cnt):
        i = pl.program_id(0)

        @pl.when(i == 0)
        def _():
            out_ref[...] = jnp.zeros_like(out_ref)
            cnt[...] = jnp.zeros_like(cnt)

        y = lax.dot_general(
            fa_ref[...], w1_ref[...], (((1,), (1,)), ((), ())),
            preferred_element_type=jnp.float32)
        y = y + lax.dot_general(
            an_ref[...], w2_ref[...], (((1,), (1,)), ((), ())),
            preferred_element_type=jnp.float32)
        hidden = jnp.maximum(y + b_ref[...], 0.0)
        ohT = (sc_ref[...] == lax.broadcasted_iota(
            jnp.int32, (blk, N_MOLS), 1)).astype(jnp.float32)
        out_ref[...] += lax.dot_general(
            ohT, hidden, (((0,), (0,)), ((), ())),
            preferred_element_type=jnp.float32)
        cnt[...] += lax.dot_general(
            ohT, jnp.ones_like(hidden), (((0,), (0,)), ((), ())),
            preferred_element_type=jnp.float32)

        @pl.when(i == nblk - 1)
        def _():
            out_ref[...] = out_ref[...] / jnp.maximum(cnt[...], 1.0)

    return pl.pallas_call(
        body,
        grid=(nblk,),
        in_specs=[
            pl.BlockSpec((blk, afd), lambda i: (i, 0)),
            pl.BlockSpec((blk, H), lambda i: (i, 0)),
            pl.BlockSpec((blk, 1), lambda i: (i, 0)),
            pl.BlockSpec((H, afd), lambda i: (0, 0)),
            pl.BlockSpec((H, H), lambda i: (0, 0)),
            pl.BlockSpec((1, H), lambda i: (0, 0)),
        ],
        out_specs=pl.BlockSpec((N_MOLS, H), lambda i: (0, 0)),
        out_shape=jax.ShapeDtypeStruct((N_MOLS, H), jnp.float32),
        scratch_shapes=[pltpu.VMEM((N_MOLS, H), jnp.float32)],
    )(fatoms, anei, scope_col, W_o1, W_o2, W_o_b)


def kernel(fatoms, fbonds, agraph, bgraph, tree_message, atom_scope,
           W_i, W_h, W_o_w, W_o_b):
    n_atoms = fatoms.shape[0]
    afd = fatoms.shape[1]

    bidx = bgraph.astype(jnp.int32).reshape(-1, NG, IG)
    n_atoms_pad = 10240
    aidx = jnp.pad(agraph.astype(jnp.int32),
                   ((0, n_atoms_pad - n_atoms), (0, 0))).reshape(-1, NG, IG)

    binput, gm = _bond_input(fbonds, W_i)
    for _ in range(DEPTH - 1):
        table = jnp.concatenate([tree_message, gm], axis=0)
        nei = _gather_sum(table, bidx, bgraph.shape[0])
        gm = _depth_update(nei, binput, W_h)

    table = jnp.concatenate([tree_message, gm], axis=0)
    anei = _gather_sum(table, aidx, n_atoms_pad)[:n_atoms]

    scope_col = atom_scope.astype(jnp.int32).reshape(-1, 1)
    W_o1 = W_o_w[:, :afd]
    W_o2 = W_o_w[:, afd:]
    b_row = W_o_b.reshape(1, H)
    return _output_layer(fatoms, anei, scope_col, W_o1, W_o2, b_row)
